# Initial kernel scaffold; baseline (speedup 1.0000x reference)
import jax
import jax.numpy as jnp
from jax.experimental import pallas as pl


def kernel(x, edge_index, batch, W1_0, b1_0, W2_0, b2_0, g_0, be_0, W1_1, b1_1, W2_1, b2_1, g_1, be_1, W1_2, b1_2, W2_2, b2_2, g_2, be_2, lw1, lb1, lw2, lb2):
    raise NotImplementedError("write your pallas kernel here")


# speedup vs baseline: 3.3806x; 3.3806x over previous
import functools

import jax
import jax.numpy as jnp
from jax import lax
from jax.experimental import pallas as pl
from jax.experimental.pallas import tpu as pltpu
from jax.experimental.pallas import tpu_sc as plsc

_N = 10000
_E = 320000
_D = 128
_G = 64
_NC = 2
_NS = 16
_NW = _NC * _NS
_CH = 128
_N_ACC = 10112
_RPT = _N_ACC // _NS

_NCHUNKS = -(-(_E // _NW) // _CH)
_EPT = _NCHUNKS * _CH
_EPAD = _NW * _EPT


def _agg_call(h, src_p, dst_p):
    mesh = plsc.VectorSubcoreMesh(core_axis_name="c", subcore_axis_name="s")

    @functools.partial(
        pl.kernel,
        out_type=jax.ShapeDtypeStruct((_NC * _N_ACC, _D), jnp.float32),
        mesh=mesh,
        scratch_types=[
            pltpu.VMEM((1, _CH), jnp.int32),
            pltpu.VMEM((1, _CH), jnp.int32),
            pltpu.VMEM((2, _CH, _D), jnp.float32),
            pltpu.VMEM_SHARED((_N_ACC, _D), jnp.float32),
            pltpu.SemaphoreType.DMA,
        ],
    )
    def k(h_hbm, src_hbm, dst_hbm, out_hbm, srci, dsti, rows, acc, sem):
        c = lax.axis_index("c")
        s = lax.axis_index("s")
        wid = s * _NC + c

        zrow = jnp.zeros((16,), jnp.float32)

        def zbody(i, carry):
            for u in range(_D // 16):
                rows[0, i, pl.ds(u * 16, 16)] = zrow
            return carry

        lax.fori_loop(0, _CH, zbody, 0)
        base = s * _RPT
        for q in range(_RPT // _CH):
            pltpu.sync_copy(rows.at[0], acc.at[pl.ds(base + q * _CH, _CH)])
        rem = _RPT % _CH
        if rem:
            pltpu.sync_copy(rows.at[0, pl.ds(0, rem)],
                            acc.at[pl.ds(base + (_RPT // _CH) * _CH, rem)])
        plsc.subcore_barrier()

        ebase = wid * _EPT

        def body(j, carry):
            off = ebase + j * _CH
            pltpu.sync_copy(src_hbm.at[pl.ds(off, _CH)], srci.at[0])
            pltpu.sync_copy(dst_hbm.at[pl.ds(off, _CH)], dsti.at[0])
            pltpu.async_copy(h_hbm.at[srci.at[0]], rows.at[0], sem).wait()
            pltpu.sync_copy(rows.at[0], acc.at[dsti.at[0]], add=True)
            return carry

        lax.fori_loop(0, _NCHUNKS, body, 0)
        plsc.subcore_barrier()

        ob = c * _N_ACC + base
        pltpu.sync_copy(acc.at[pl.ds(base, _RPT)], out_hbm.at[pl.ds(ob, _RPT)])

    return k(h, src_p, dst_p)


def _mlp_body(h_ref, a0_ref, a1_ref, w1_ref, b1_ref, w2_ref, b2_ref,
              g_ref, be_ref, o_ref):
    z = h_ref[...] + a0_ref[...] + a1_ref[...]
    z = jnp.maximum(
        jnp.dot(z, w1_ref[...], preferred_element_type=jnp.float32)
        + b1_ref[...], 0.0)
    z = jnp.maximum(
        jnp.dot(z, w2_ref[...], preferred_element_type=jnp.float32)
        + b2_ref[...], 0.0)
    m = jnp.mean(z, axis=0, keepdims=True)
    v = jnp.mean((z - m) ** 2, axis=0, keepdims=True)
    o_ref[...] = (z - m) / jnp.sqrt(v + 1e-5) * g_ref[...] + be_ref[...]


def _mlp_call(h, a0, a1, w1, b1, w2, b2, g, be):
    return pl.pallas_call(
        _mlp_body,
        out_shape=jax.ShapeDtypeStruct((_N, _D), jnp.float32),
    )(h, a0, a1, w1, b1.reshape(1, _D), w2, b2.reshape(1, _D),
      g.reshape(1, _D), be.reshape(1, _D))


def _final_body(h_ref, bidx_ref, lw1_ref, lb1_ref, lw2_ref, lb2_ref, o_ref):
    h = h_ref[...]
    bidx = bidx_ref[...]
    rid = lax.broadcasted_iota(jnp.int32, (_G, 1), 0)
    neg = jnp.float32(-jnp.inf)

    def body(gi, carry):
        col = jnp.max(jnp.where(bidx == gi, h, neg), axis=0, keepdims=True)
        return jnp.where(rid == gi, col, carry)

    pooled = lax.fori_loop(0, _G, body,
                           jnp.full((_G, _D), neg, jnp.float32))
    r = jnp.maximum(
        jnp.dot(pooled, lw1_ref[...], preferred_element_type=jnp.float32)
        + lb1_ref[...], 0.0)
    o_ref[...] = (jnp.dot(r, lw2_ref[...], preferred_element_type=jnp.float32)
                  + lb2_ref[...])


def _final_call(h, batch, lw1, lb1, lw2, lb2):
    c = lw2.shape[1]
    lw2p = jnp.zeros((_D, _D), jnp.float32).at[:, :c].set(lw2)
    lb2p = jnp.zeros((1, _D), jnp.float32).at[:, :c].set(lb2.reshape(1, c))
    out = pl.pallas_call(
        _final_body,
        out_shape=jax.ShapeDtypeStruct((_G, _D), jnp.float32),
    )(h, batch.reshape(_N, 1), lw1, lb1.reshape(1, _D), lw2p, lb2p)
    return out[:, :c]


def kernel(x, edge_index, batch,
           W1_0, b1_0, W2_0, b2_0, g_0, be_0,
           W1_1, b1_1, W2_1, b2_1, g_1, be_1,
           W1_2, b1_2, W2_2, b2_2, g_2, be_2,
           lw1, lb1, lw2, lb2):
    src = edge_index[0]
    dst = edge_index[1]
    npad = _EPAD - _E
    src_p = jnp.concatenate([src, jnp.zeros((npad,), jnp.int32)])
    dst_p = jnp.concatenate([dst, jnp.full((npad,), _N, jnp.int32)])

    layers = [(W1_0, b1_0, W2_0, b2_0, g_0, be_0),
              (W1_1, b1_1, W2_1, b2_1, g_1, be_1),
              (W1_2, b1_2, W2_2, b2_2, g_2, be_2)]
    h = x
    for (w1, b1, w2, b2, g, be) in layers:
        agg = _agg_call(h, src_p, dst_p)
        a0 = agg[:_N]
        a1 = agg[_N_ACC:_N_ACC + _N]
        h = _mlp_call(h, a0, a1, w1, b1, w2, b2, g, be)
    return _final_call(h, batch, lw1, lb1, lw2, lb2)
